# baseline (speedup 1.0000x reference)
import functools

import jax
import jax.numpy as jnp
from jax import lax
from jax.experimental import pallas as pl
from jax.experimental.pallas import tpu as pltpu
from jax.experimental.pallas import tpu_sc as plsc

NUM_EMB = 1_000_000
EMBED_DIM = 32
BATCH = 16384
HIST = 200

NW = 32
ROWS_PER_W = BATCH // NW
CHUNK_ROWS = 128
N_CHUNKS = ROWS_PER_W // CHUNK_ROWS
IDX_PER_CHUNK = CHUNK_ROWS * HIST
IDX_ROWS = IDX_PER_CHUNK // 128
XROWS_PER_W = (BATCH * HIST) // 128 // NW

TBLOCK = 12712
TSTEPS = 20
TC_ROWS = 2 * TBLOCK * TSTEPS
SC_ROWS = NUM_EMB - TC_ROWS
SROWS_PER_W = SC_ROWS // NW
SCHUNK = 384
SCHUNKS_PER_W = SROWS_PER_W // SCHUNK


def _table_body(emb_a_ref, emb_b_ref, w_ref, b_ref, out_ref):
    for h, ref in enumerate((emb_a_ref, emb_b_ref)):
        out_ref[0, 0, pl.ds(h * TBLOCK, TBLOCK)] = (
            lax.dot_general(
                w_ref[...],
                ref[...],
                dimension_numbers=(((1,), (1,)), ((), ())),
                preferred_element_type=jnp.float32,
            )[0]
            + b_ref[0]
        )


def _make_table_tc(emb, w_scaled, b_scaled):
    return pl.pallas_call(
        _table_body,
        grid=(TSTEPS,),
        in_specs=[
            pl.BlockSpec((TBLOCK, EMBED_DIM), lambda i: (2 * i, 0)),
            pl.BlockSpec((TBLOCK, EMBED_DIM), lambda i: (2 * i + 1, 0)),
            pl.BlockSpec((1, EMBED_DIM), lambda i: (0, 0)),
            pl.BlockSpec(memory_space=pltpu.SMEM),
        ],
        out_specs=pl.BlockSpec((1, 1, 2 * TBLOCK), lambda i: (i, 0, 0)),
        out_shape=jax.ShapeDtypeStruct((TSTEPS, 1, 2 * TBLOCK), jnp.float32),
    )(emb, emb, w_scaled, b_scaled)


def _make_table_sc(emb, wb):
    mesh = plsc.VectorSubcoreMesh(core_axis_name="c", subcore_axis_name="s")

    @functools.partial(
        pl.kernel,
        out_type=jax.ShapeDtypeStruct((SC_ROWS,), jnp.float32),
        mesh=mesh,
        compiler_params=pltpu.CompilerParams(needs_layout_passes=False),
        scratch_types=[
            pltpu.VMEM((2, SCHUNK, EMBED_DIM), jnp.float32),
            pltpu.VMEM((SCHUNK,), jnp.float32),
            pltpu.VMEM((48,), jnp.float32),
            pltpu.SemaphoreType.DMA,
            pltpu.SemaphoreType.DMA,
        ],
    )
    def body(emb_hbm, wb_hbm, p_hbm, buf_v, out_v, wb_v, sem0, sem1):
        wid = lax.axis_index("c") * 16 + lax.axis_index("s")
        base = TC_ROWS + wid * SROWS_PER_W
        pltpu.sync_copy(wb_hbm, wb_v)
        w_lo = wb_v[pl.ds(0, 16)]
        w_hi = wb_v[pl.ds(16, 16)]
        b_sc = wb_v[pl.ds(32, 16)][0]
        iota16 = lax.iota(jnp.int32, 16)
        sems = (sem0, sem1)

        def start(c):
            for pb in range(2):
                @pl.when(jnp.logical_and(c < SCHUNKS_PER_W, lax.rem(c, 2) == pb))
                def _():
                    pltpu.async_copy(
                        emb_hbm.at[pl.ds(base + c * SCHUNK, SCHUNK)],
                        buf_v.at[pb],
                        sems[pb],
                    )

        def drain(c):
            for pb in range(2):
                @pl.when(lax.rem(c, 2) == pb)
                def _():
                    pltpu.make_async_copy(
                        emb_hbm.at[pl.ds(0, SCHUNK)], buf_v.at[pb], sems[pb]
                    ).wait()

        def compute(c):
            for pb in range(2):
                @pl.when(lax.rem(c, 2) == pb)
                def _():
                    def group(g, carry):
                        i0 = g * 16
                        acc = jnp.zeros((16,), jnp.float32) + b_sc
                        for d in range(EMBED_DIM):
                            wd = w_lo[d] if d < 16 else w_hi[d - 16]
                            acc = acc + wd * plsc.load_gather(
                                buf_v.at[pb],
                                [i0 + iota16, jnp.full((16,), d, jnp.int32)],
                            )
                        out_v[pl.ds(pl.multiple_of(g * 16, 16), 16)] = acc
                        return carry

                    lax.fori_loop(0, SCHUNK // 16, group, 0)
            pltpu.sync_copy(
                out_v,
                p_hbm.at[pl.ds(wid * SROWS_PER_W + c * SCHUNK, SCHUNK)],
            )

        start(jnp.int32(0))

        def step(c, carry):
            start(c + 1)
            drain(c)
            compute(c)
            return carry

        lax.fori_loop(0, SCHUNKS_PER_W, step, 0)

    return body(emb, wb)


def _gather_sum(x2, p):
    mesh = plsc.VectorSubcoreMesh(core_axis_name="c", subcore_axis_name="s")

    @functools.partial(
        pl.kernel,
        out_type=jax.ShapeDtypeStruct((BATCH,), jnp.float32),
        mesh=mesh,
        compiler_params=pltpu.CompilerParams(needs_layout_passes=False),
        scratch_types=[
            pltpu.VMEM((IDX_ROWS, 128), jnp.int32),
            pltpu.VMEM((IDX_PER_CHUNK,), jnp.float32),
            pltpu.VMEM((ROWS_PER_W,), jnp.float32),
            pltpu.SemaphoreType.DMA,
        ],
    )
    def body(x2_hbm, p_hbm, y_hbm, idx_v, vals_v, out_v, sem):
        wid = lax.axis_index("c") * 16 + lax.axis_index("s")
        iota200 = lax.iota(jnp.int32, 16) * HIST

        def chunk_body(c, carry):
            xrow = wid * XROWS_PER_W + c * IDX_ROWS
            pltpu.sync_copy(x2_hbm.at[pl.ds(xrow, IDX_ROWS)], idx_v)

            def fire(j, carry2):
                for k in range(8):
                    r = j * 8 + k
                    pltpu.async_copy(
                        p_hbm.at[idx_v.at[r]],
                        vals_v.at[pl.ds(pl.multiple_of(r * 128, 128), 128)],
                        sem,
                    )
                return carry2

            lax.fori_loop(0, IDX_ROWS // 8, fire, 0)
            pltpu.make_async_copy(
                p_hbm.at[pl.ds(0, IDX_PER_CHUNK)], vals_v, sem
            ).wait()

            for g in range(8):
                base = g * 16 * HIST

                def red(j, acc):
                    jb = base + j * 8
                    vs = [
                        plsc.load_gather(vals_v, [iota200 + (jb + k)])
                        for k in range(8)
                    ]
                    s = ((vs[0] + vs[1]) + (vs[2] + vs[3])) + (
                        (vs[4] + vs[5]) + (vs[6] + vs[7])
                    )
                    return acc + s

                acc = lax.fori_loop(
                    0, HIST // 8, red, jnp.zeros((16,), jnp.float32)
                )
                out_v[pl.ds(pl.multiple_of(c * CHUNK_ROWS + g * 16, 16), 16)] = acc
            return carry

        lax.fori_loop(0, N_CHUNKS, chunk_body, 0)
        pltpu.sync_copy(
            out_v, y_hbm.at[pl.ds(pl.multiple_of(wid * ROWS_PER_W, 512), ROWS_PER_W)]
        )

    return body(x2, p)


def kernel(X, emb, W, b):
    x2 = X.astype(jnp.int32).reshape(BATCH * HIST // 128, 128)
    w_scaled = W.astype(jnp.float32).reshape(1, EMBED_DIM) * (1.0 / HIST)
    b_scaled = b.astype(jnp.float32).reshape(1) * (1.0 / HIST)
    wb = jnp.zeros((48,), jnp.float32)
    wb = lax.dynamic_update_slice(wb, w_scaled.reshape(EMBED_DIM), (0,))
    wb = lax.dynamic_update_slice(wb, b_scaled, (32,))
    p_hi = _make_table_sc(emb, wb)
    p_lo = _make_table_tc(emb, w_scaled, b_scaled).reshape(TC_ROWS)
    p = jnp.concatenate([p_lo, p_hi])
    y = _gather_sum(x2, p)
    return y.reshape(BATCH, 1)


# speedup vs baseline: 1.0179x; 1.0179x over previous
import functools

import jax
import jax.numpy as jnp
from jax import lax
from jax.experimental import pallas as pl
from jax.experimental.pallas import tpu as pltpu
from jax.experimental.pallas import tpu_sc as plsc

NUM_EMB = 1_000_000
EMBED_DIM = 32
BATCH = 16384
HIST = 200

NW = 32
ROWS_PER_W = BATCH // NW
CHUNK_ROWS = 128
N_CHUNKS = ROWS_PER_W // CHUNK_ROWS
IDX_PER_CHUNK = CHUNK_ROWS * HIST
IDX_ROWS = IDX_PER_CHUNK // 128
XROWS_PER_W = (BATCH * HIST) // 128 // NW

TBLOCK = 20000


def _table_body(emb_a_ref, emb_b_ref, w_ref, b_ref, out_ref):
    for h, ref in enumerate((emb_a_ref, emb_b_ref)):
        out_ref[0, 0, pl.ds(h * TBLOCK, TBLOCK)] = (
            lax.dot_general(
                w_ref[...],
                ref[...],
                dimension_numbers=(((1,), (1,)), ((), ())),
                preferred_element_type=jnp.float32,
            )[0]
            + b_ref[0]
        )


def _make_table(emb, w_scaled, b_scaled):
    grid = (NUM_EMB // (2 * TBLOCK),)
    return pl.pallas_call(
        _table_body,
        grid=grid,
        in_specs=[
            pl.BlockSpec((TBLOCK, EMBED_DIM), lambda i: (2 * i, 0)),
            pl.BlockSpec((TBLOCK, EMBED_DIM), lambda i: (2 * i + 1, 0)),
            pl.BlockSpec((1, EMBED_DIM), lambda i: (0, 0)),
            pl.BlockSpec(memory_space=pltpu.SMEM),
        ],
        out_specs=pl.BlockSpec((1, 1, 2 * TBLOCK), lambda i: (i, 0, 0)),
        out_shape=jax.ShapeDtypeStruct(
            (NUM_EMB // (2 * TBLOCK), 1, 2 * TBLOCK), jnp.float32
        ),
    )(emb, emb, w_scaled, b_scaled)


def _gather_sum(x2, p):
    mesh = plsc.VectorSubcoreMesh(core_axis_name="c", subcore_axis_name="s")

    @functools.partial(
        pl.kernel,
        out_type=jax.ShapeDtypeStruct((BATCH,), jnp.float32),
        mesh=mesh,
        compiler_params=pltpu.CompilerParams(needs_layout_passes=False),
        scratch_types=[
            pltpu.VMEM((IDX_ROWS, 128), jnp.int32),
            pltpu.VMEM((IDX_ROWS, 128), jnp.int32),
            pltpu.VMEM((IDX_PER_CHUNK,), jnp.float32),
            pltpu.VMEM((IDX_PER_CHUNK,), jnp.float32),
            pltpu.VMEM((ROWS_PER_W,), jnp.float32),
            pltpu.SemaphoreType.DMA,
            pltpu.SemaphoreType.DMA,
            pltpu.SemaphoreType.DMA,
            pltpu.SemaphoreType.DMA,
        ],
    )
    def body(x2_hbm, p_hbm, y_hbm, idx0_v, idx1_v, vals0_v, vals1_v, out_v,
             isem0, isem1, vsem0, vsem1):
        wid = lax.axis_index("c") * 16 + lax.axis_index("s")
        iota200 = lax.iota(jnp.int32, 16) * HIST
        isems = (isem0, isem1)
        vsems = (vsem0, vsem1)
        idxs = (idx0_v, idx1_v)
        valss = (vals0_v, vals1_v)

        def start_idx(c):
            for pb in range(2):
                @pl.when(jnp.logical_and(c < N_CHUNKS, lax.rem(c, 2) == pb))
                def _():
                    xrow = wid * XROWS_PER_W + c * IDX_ROWS
                    pltpu.async_copy(
                        x2_hbm.at[pl.ds(xrow, IDX_ROWS)], idxs[pb],
                        isems[pb],
                    )

        def wait_idx(c):
            for pb in range(2):
                @pl.when(jnp.logical_and(c < N_CHUNKS, lax.rem(c, 2) == pb))
                def _():
                    pltpu.make_async_copy(
                        x2_hbm.at[pl.ds(0, IDX_ROWS)], idxs[pb], isems[pb]
                    ).wait()

        def fire(c):
            for pb in range(2):
                @pl.when(jnp.logical_and(c < N_CHUNKS, lax.rem(c, 2) == pb))
                def _():
                    def fire_j(j, carry2):
                        for k in range(8):
                            r = j * 8 + k
                            pltpu.async_copy(
                                p_hbm.at[idxs[pb].at[r]],
                                valss[pb].at[
                                    pl.ds(pl.multiple_of(r * 128, 128), 128)
                                ],
                                vsems[pb],
                            )
                        return carry2

                    lax.fori_loop(0, IDX_ROWS // 8, fire_j, 0)

        def drain(c):
            for pb in range(2):
                @pl.when(lax.rem(c, 2) == pb)
                def _():
                    pltpu.make_async_copy(
                        p_hbm.at[pl.ds(0, IDX_PER_CHUNK)], valss[pb],
                        vsems[pb],
                    ).wait()

        def reduce(c):
            for pb in range(2):
                @pl.when(lax.rem(c, 2) == pb)
                def _():
                    for g in range(8):
                        base = g * 16 * HIST

                        def red(j, acc):
                            jb = base + j * 8
                            vs = [
                                plsc.load_gather(
                                    valss[pb], [iota200 + (jb + k)]
                                )
                                for k in range(8)
                            ]
                            s = ((vs[0] + vs[1]) + (vs[2] + vs[3])) + (
                                (vs[4] + vs[5]) + (vs[6] + vs[7])
                            )
                            return acc + s

                        acc = lax.fori_loop(
                            0, HIST // 8, red, jnp.zeros((16,), jnp.float32)
                        )
                        out_v[
                            pl.ds(pl.multiple_of(c * CHUNK_ROWS + g * 16, 16), 16)
                        ] = acc

        start_idx(jnp.int32(0))
        start_idx(jnp.int32(1))
        wait_idx(jnp.int32(0))
        fire(jnp.int32(0))

        def step(c, carry):
            wait_idx(c + 1)
            fire(c + 1)
            drain(c)
            start_idx(c + 2)
            reduce(c)
            return carry

        lax.fori_loop(0, N_CHUNKS, step, 0)
        pltpu.sync_copy(
            out_v, y_hbm.at[pl.ds(pl.multiple_of(wid * ROWS_PER_W, 512), ROWS_PER_W)]
        )

    return body(x2, p)


def kernel(X, emb, W, b):
    x2 = X.astype(jnp.int32).reshape(BATCH * HIST // 128, 128)
    w_scaled = W.astype(jnp.float32).reshape(1, EMBED_DIM) * (1.0 / HIST)
    b_scaled = b.astype(jnp.float32).reshape(1) * (1.0 / HIST)
    p = _make_table(emb, w_scaled, b_scaled).reshape(NUM_EMB)
    y = _gather_sum(x2, p)
    return y.reshape(BATCH, 1)


# speedup vs baseline: 1.1456x; 1.1255x over previous
import functools

import jax
import jax.numpy as jnp
from jax import lax
from jax.experimental import pallas as pl
from jax.experimental.pallas import tpu as pltpu
from jax.experimental.pallas import tpu_sc as plsc

NUM_EMB = 1_000_000
EMBED_DIM = 32
BATCH = 16384
HIST = 200

NW = 32
ROWS_PER_W = BATCH // NW
CHUNK_ROWS = 64
N_CHUNKS = ROWS_PER_W // CHUNK_ROWS
IDX_PER_CHUNK = CHUNK_ROWS * HIST
IDX_ROWS = IDX_PER_CHUNK // 128
XROWS_PER_W = (BATCH * HIST) // 128 // NW

TBLOCK = 20000


def _table_body(emb_a_ref, emb_b_ref, w_ref, b_ref, out_ref):
    for h, ref in enumerate((emb_a_ref, emb_b_ref)):
        out_ref[0, 0, pl.ds(h * TBLOCK, TBLOCK)] = (
            lax.dot_general(
                w_ref[...],
                ref[...],
                dimension_numbers=(((1,), (1,)), ((), ())),
                preferred_element_type=jnp.float32,
            )[0]
            + b_ref[0]
        )


def _make_table(emb, w_scaled, b_scaled):
    grid = (NUM_EMB // (2 * TBLOCK),)
    return pl.pallas_call(
        _table_body,
        grid=grid,
        in_specs=[
            pl.BlockSpec((TBLOCK, EMBED_DIM), lambda i: (2 * i, 0)),
            pl.BlockSpec((TBLOCK, EMBED_DIM), lambda i: (2 * i + 1, 0)),
            pl.BlockSpec((1, EMBED_DIM), lambda i: (0, 0)),
            pl.BlockSpec(memory_space=pltpu.SMEM),
        ],
        out_specs=pl.BlockSpec((1, 1, 2 * TBLOCK), lambda i: (i, 0, 0)),
        out_shape=jax.ShapeDtypeStruct(
            (NUM_EMB // (2 * TBLOCK), 1, 2 * TBLOCK), jnp.float32
        ),
    )(emb, emb, w_scaled, b_scaled)


def _gather_sum(x2, p):
    mesh = plsc.VectorSubcoreMesh(core_axis_name="c", subcore_axis_name="s")

    @functools.partial(
        pl.kernel,
        out_type=jax.ShapeDtypeStruct((BATCH,), jnp.float32),
        mesh=mesh,
        compiler_params=pltpu.CompilerParams(needs_layout_passes=False),
        scratch_types=[
            pltpu.VMEM((200, 128), jnp.int32),
            pltpu.VMEM((IDX_PER_CHUNK,), jnp.float32),
            pltpu.VMEM((25000,), jnp.float32),
            pltpu.VMEM((ROWS_PER_W,), jnp.float32),
            pltpu.VMEM_SHARED((NUM_EMB,), jnp.float32),
            pltpu.SemaphoreType.DMA,
        ],
    )
    def body(x2_hbm, p_hbm, y_hbm, idx_v, vals_v, bounce_v, out_v, p_sh, sem):
        wid = lax.axis_index("c") * 16 + lax.axis_index("s")
        iota200 = lax.iota(jnp.int32, 16) * HIST

        sid = lax.axis_index("s")
        @pl.when(sid < 8)
        def _():
            for t in range(5):
                off = pl.multiple_of(sid * (NUM_EMB // 8) + t * 25000, 8)
                pltpu.sync_copy(p_hbm.at[pl.ds(off, 25000)], bounce_v)
                pltpu.sync_copy(bounce_v, p_sh.at[pl.ds(off, 25000)])
        plsc.subcore_barrier()

        def chunk_body(c, carry):
            half = lax.rem(c, 2)

            @pl.when(half == 0)
            def _():
                xrow = wid * XROWS_PER_W + (c // 2) * 200
                pltpu.sync_copy(x2_hbm.at[pl.ds(xrow, 200)], idx_v)

            def fire_j(j, carry2):
                for k in range(4):
                    r = j * 4 + k
                    pltpu.async_copy(
                        p_sh.at[idx_v.at[half * IDX_ROWS + r]],
                        vals_v.at[pl.ds(pl.multiple_of(r * 128, 128), 128)],
                        sem,
                    )
                return carry2

            lax.fori_loop(0, IDX_ROWS // 4, fire_j, 0)
            pltpu.make_async_copy(
                p_hbm.at[pl.ds(0, IDX_PER_CHUNK)], vals_v, sem
            ).wait()

            for g in range(CHUNK_ROWS // 16):
                base = g * 16 * HIST

                def red(j, acc):
                    jb = base + j * 8
                    vs = [
                        plsc.load_gather(vals_v, [iota200 + (jb + k)])
                        for k in range(8)
                    ]
                    s = ((vs[0] + vs[1]) + (vs[2] + vs[3])) + (
                        (vs[4] + vs[5]) + (vs[6] + vs[7])
                    )
                    return acc + s

                acc = lax.fori_loop(
                    0, HIST // 8, red, jnp.zeros((16,), jnp.float32)
                )
                out_v[
                    pl.ds(pl.multiple_of(c * CHUNK_ROWS + g * 16, 16), 16)
                ] = acc
            return carry

        lax.fori_loop(0, N_CHUNKS, chunk_body, 0)
        pltpu.sync_copy(
            out_v, y_hbm.at[pl.ds(pl.multiple_of(wid * ROWS_PER_W, 512), ROWS_PER_W)]
        )

    return body(x2, p)


def kernel(X, emb, W, b):
    x2 = X.astype(jnp.int32).reshape(BATCH * HIST // 128, 128)
    w_scaled = W.astype(jnp.float32).reshape(1, EMBED_DIM) * (1.0 / HIST)
    b_scaled = b.astype(jnp.float32).reshape(1) * (1.0 / HIST)
    p = _make_table(emb, w_scaled, b_scaled).reshape(NUM_EMB)
    y = _gather_sum(x2, p)
    return y.reshape(BATCH, 1)
